# baseline (speedup 1.0000x reference)
import numpy as np
import jax
import jax.numpy as jnp
from jax.experimental import pallas as pl
from jax.experimental.pallas import tpu as pltpu

_N = 20000
_NP = 20480
_ROWS = 160
_PRE_K = 1000
_POST_K = 100
_SCORE_THRESH = 0.05
_NMS_THRESH = 0.6
_P = 1024
_SENT = -1.0e30


def _full_kernel(boxes_ref, scol_ref, s2d_ref, out_ref, pos_s, acc_s, iou_s):
    f32 = jnp.float32
    i32 = jnp.int32

    s2d = s2d_ref[:, :]
    masked = jnp.where(s2d > _SCORE_THRESH, s2d, _SENT)
    bits = jax.lax.bitcast_convert_type(masked, i32)
    bits = jnp.where(s2d > _SCORE_THRESH, bits, -1)

    def vsearch(_, lohi):
        lo, hi = lohi
        mid = lo + (hi - lo) // 2
        cnt = jnp.sum((bits >= mid).astype(i32))
        big = cnt >= _PRE_K
        return jnp.where(big, mid, lo), jnp.where(big, hi, mid)

    vstar, _ = jax.lax.fori_loop(
        0, 32, vsearch, (jnp.int32(-1), jnp.int32(0x40000000)))

    sub_r = jax.lax.broadcasted_iota(i32, (_ROWS, 128), 0)
    lan_r = jax.lax.broadcasted_iota(i32, (_ROWS, 128), 1)
    gidx = sub_r * 128 + lan_r
    m_need = _PRE_K - jnp.sum((bits > vstar).astype(i32))

    def isearch(_, lohi):
        lo, hi = lohi
        mid = lo + (hi - lo) // 2
        cnt = jnp.sum(((bits == vstar) & (gidx < mid)).astype(i32))
        big = cnt >= m_need
        return jnp.where(big, lo, mid), jnp.where(big, mid, hi)

    _, icut = jax.lax.fori_loop(
        0, 16, isearch, (jnp.int32(0), jnp.int32(_NP)))

    sel = (bits > vstar) | ((bits == vstar) & (gidx < icut))
    sel_f = sel.astype(f32)

    tsub = jax.lax.broadcasted_iota(i32, (128, 128), 0)
    tlan = jax.lax.broadcasted_iota(i32, (128, 128), 1)
    tri128 = (tsub <= tlan).astype(f32)
    rowcum = jax.lax.dot_general(
        sel_f, tri128, (((1,), (0,)), ((), ())),
        preferred_element_type=f32, precision=jax.lax.Precision.HIGHEST)
    rowtot = rowcum[:, 127:128]
    ssub = jax.lax.broadcasted_iota(i32, (_ROWS, _ROWS), 0)
    slan = jax.lax.broadcasted_iota(i32, (_ROWS, _ROWS), 1)
    strict = (slan < ssub).astype(f32)
    pfx = jax.lax.dot_general(
        strict, rowtot, (((1,), (0,)), ((), ())),
        preferred_element_type=f32, precision=jax.lax.Precision.HIGHEST)
    pos_s[:, :] = jnp.where(sel, pfx + rowcum - 1.0, -1.0)

    acc_s[:, :] = jnp.zeros((_P, 8), f32)
    sub_p1 = jax.lax.broadcasted_iota(i32, (_P, 1), 0).astype(f32)
    sub_c1 = jax.lax.broadcasted_iota(i32, (128, 1), 0).astype(f32)

    def gather_chunk(c, carry):
        prow = pos_s[pl.ds(c, 1), :]
        b = boxes_ref[pl.ds(c * 128, 128), :]
        sc = scol_ref[pl.ds(c * 128, 128), :]
        msc = jnp.where(sc > _SCORE_THRESH, sc, _SENT)
        gi = sub_c1 + jnp.float32(128) * c.astype(f32)
        d = jnp.concatenate([b, msc, gi, jnp.zeros((128, 2), f32)], axis=1)
        onehot = (sub_p1 == prow).astype(f32)
        acc_s[:, :] += jax.lax.dot_general(
            onehot, d, (((1,), (0,)), ((), ())),
            preferred_element_type=f32, precision=jax.lax.Precision.HIGHEST)
        return carry

    jax.lax.fori_loop(0, _ROWS, gather_chunk, jnp.int32(0))

    psub = jax.lax.broadcasted_iota(i32, (_P, _P), 0)
    plan = jax.lax.broadcasted_iota(i32, (_P, _P), 1)
    eye = (psub == plan).astype(f32)
    compact = acc_s[:, :]
    compact_t = jax.lax.dot_general(
        compact, eye, (((0,), (0,)), ((), ())),
        preferred_element_type=f32, precision=jax.lax.Precision.HIGHEST)
    s_col = compact[:, 4:5]
    i_col = compact[:, 5:6]
    s_row = compact_t[4:5, :]
    i_row = compact_t[5:6, :]
    before = ((s_col > s_row) | ((s_col == s_row) & (i_col < i_row))).astype(f32)
    rank = jax.lax.dot_general(
        jnp.ones((1, _P), f32), before, (((1,), (0,)), ((), ())),
        preferred_element_type=f32, precision=jax.lax.Precision.HIGHEST)
    perm = (sub_p1 == rank).astype(f32)
    sorted_d = jax.lax.dot_general(
        perm, compact, (((1,), (0,)), ((), ())),
        preferred_element_type=f32, precision=jax.lax.Precision.HIGHEST)
    sorted_t = jax.lax.dot_general(
        sorted_d, eye, (((0,), (0,)), ((), ())),
        preferred_element_type=f32, precision=jax.lax.Precision.HIGHEST)

    x1c, y1c = sorted_d[:, 0:1], sorted_d[:, 1:2]
    x2c, y2c = sorted_d[:, 2:3], sorted_d[:, 3:4]
    x1r, y1r = sorted_t[0:1, :], sorted_t[1:2, :]
    x2r, y2r = sorted_t[2:3, :], sorted_t[3:4, :]
    area_c = jnp.maximum(x2c - x1c, 0.0) * jnp.maximum(y2c - y1c, 0.0)
    area_r = jnp.maximum(x2r - x1r, 0.0) * jnp.maximum(y2r - y1r, 0.0)
    iw = jnp.maximum(jnp.minimum(x2c, x2r) - jnp.maximum(x1c, x1r), 0.0)
    ih = jnp.maximum(jnp.minimum(y2c, y2r) - jnp.maximum(y1c, y1r), 0.0)
    inter = iw * ih
    union = area_c + area_r - inter
    iou_s[:, :] = inter / jnp.maximum(union, 1e-9)

    lane1 = jax.lax.broadcasted_iota(i32, (1, _P), 1)

    def body(i, keep):
        row = iou_s[pl.ds(i, 1), :]
        keep_i = jnp.sum(jnp.where(lane1 == i, keep, 0.0))
        sup = (row > _NMS_THRESH) & (lane1 > i) & (keep_i > 0.5)
        return jnp.where(sup, 0.0, keep)

    keep = jax.lax.fori_loop(0, _PRE_K, body, jnp.ones((1, _P), f32))

    srow_s = sorted_t[4:5, :]
    k_row = keep * (srow_s > _SCORE_THRESH).astype(f32)
    tri_p = (psub <= plan).astype(f32)
    cum_k = jax.lax.dot_general(
        k_row, tri_p, (((1,), (0,)), ((), ())), preferred_element_type=f32, precision=jax.lax.Precision.HIGHEST)
    cum_nk = jax.lax.dot_general(
        1.0 - k_row, tri_p, (((1,), (0,)), ((), ())),
        preferred_element_type=f32, precision=jax.lax.Precision.HIGHEST)
    num_k = jnp.sum(k_row)
    slot = jnp.where(k_row > 0.5, cum_k - 1.0, num_k + cum_nk - 1.0)
    out_row = jax.lax.broadcasted_iota(i32, (128, 1), 0).astype(f32)
    sel_m = (out_row == slot).astype(f32)
    k_col = jax.lax.dot_general(
        eye, k_row, (((1,), (1,)), ((), ())), preferred_element_type=f32, precision=jax.lax.Precision.HIGHEST)
    s_keep = jnp.where(k_col > 0.5, sorted_d[:, 4:5], 0.0)
    data = jnp.concatenate(
        [sorted_d[:, 0:4], s_keep, jnp.zeros((_P, 3), f32)], axis=1)
    out_ref[:, :] = jax.lax.dot_general(
        sel_m, data, (((1,), (0,)), ((), ())), preferred_element_type=f32, precision=jax.lax.Precision.HIGHEST)


def kernel(boxes, scores):
    pad = _NP - _N
    boxes_p = jnp.concatenate([boxes, jnp.zeros((pad, 4), jnp.float32)], axis=0)
    scores_p = jnp.concatenate([scores, jnp.zeros((pad,), jnp.float32)], axis=0)
    scol = scores_p[:, None]
    s2d = scores_p.reshape(_ROWS, 128)

    out = pl.pallas_call(
        _full_kernel,
        out_shape=jax.ShapeDtypeStruct((128, 8), jnp.float32),
        scratch_shapes=[
            pltpu.VMEM((_ROWS, 128), jnp.float32),
            pltpu.VMEM((_P, 8), jnp.float32),
            pltpu.VMEM((_P, _P), jnp.float32),
        ],
    )(boxes_p, scol, s2d)
    return out[:_POST_K, :5]


# speedup vs baseline: 1.2663x; 1.2663x over previous
import numpy as np
import jax
import jax.numpy as jnp
from jax.experimental import pallas as pl
from jax.experimental.pallas import tpu as pltpu

_N = 20000
_NP = 20480
_ROWS = 160
_PRE_K = 1000
_POST_K = 100
_SCORE_THRESH = 0.05
_NMS_THRESH = 0.6
_P = 1024
_SENT = -1.0e30


def _full_kernel(boxes_ref, scol_ref, s2d_ref, out_ref, pos_s, acc_s, iou_s):
    f32 = jnp.float32
    i32 = jnp.int32

    s2d = s2d_ref[:, :]
    masked = jnp.where(s2d > _SCORE_THRESH, s2d, _SENT)
    bits = jax.lax.bitcast_convert_type(masked, i32)
    bits = jnp.where(s2d > _SCORE_THRESH, bits, -1)

    def vsearch(_, lohi):
        lo, hi = lohi
        mid = lo + (hi - lo) // 2
        cnt = jnp.sum((bits >= mid).astype(i32))
        big = cnt >= _PRE_K
        return jnp.where(big, mid, lo), jnp.where(big, hi, mid)

    vstar, _ = jax.lax.fori_loop(
        0, 32, vsearch, (jnp.int32(-1), jnp.int32(0x40000000)))

    sub_r = jax.lax.broadcasted_iota(i32, (_ROWS, 128), 0)
    lan_r = jax.lax.broadcasted_iota(i32, (_ROWS, 128), 1)
    gidx = sub_r * 128 + lan_r
    m_need = _PRE_K - jnp.sum((bits > vstar).astype(i32))

    def isearch(_, lohi):
        lo, hi = lohi
        mid = lo + (hi - lo) // 2
        cnt = jnp.sum(((bits == vstar) & (gidx < mid)).astype(i32))
        big = cnt >= m_need
        return jnp.where(big, lo, mid), jnp.where(big, mid, hi)

    _, icut = jax.lax.fori_loop(
        0, 16, isearch, (jnp.int32(0), jnp.int32(_NP)))

    sel = (bits > vstar) | ((bits == vstar) & (gidx < icut))
    sel_f = sel.astype(f32)

    tsub = jax.lax.broadcasted_iota(i32, (128, 128), 0)
    tlan = jax.lax.broadcasted_iota(i32, (128, 128), 1)
    tri128 = (tsub <= tlan).astype(f32)
    rowcum = jax.lax.dot_general(
        sel_f, tri128, (((1,), (0,)), ((), ())),
        preferred_element_type=f32, precision=jax.lax.Precision.HIGHEST)
    rowtot = rowcum[:, 127:128]
    pos_s[:, :] = jnp.where(sel, rowcum - 1.0, -1.0)

    sub_p1 = jax.lax.broadcasted_iota(i32, (_P, 1), 0).astype(f32)
    sub_c1 = jax.lax.broadcasted_iota(i32, (128, 1), 0).astype(f32)
    sub_r1 = jax.lax.broadcasted_iota(i32, (_ROWS, 1), 0)

    def gather_chunk(c, carry):
        prow = pos_s[pl.ds(c, 1), :]
        b = boxes_ref[pl.ds(c * 128, 128), :]
        sc = scol_ref[pl.ds(c * 128, 128), :]
        msc = jnp.where(sc > _SCORE_THRESH, sc, _SENT)
        gi = sub_c1 + jnp.float32(128) * c.astype(f32)
        d = jnp.concatenate([b, msc, gi, jnp.zeros((128, 2), f32)], axis=1)
        onehot = (sub_c1 == prow).astype(f32)
        dcomp = jax.lax.dot_general(
            onehot, d, (((1,), (0,)), ((), ())),
            preferred_element_type=f32, precision=jax.lax.Precision.HIGHEST)
        base = jnp.sum(jnp.where(sub_r1 < c, rowtot, 0.0)).astype(i32)
        acc_s[pl.ds(base, 128), :] = dcomp
        return carry

    jax.lax.fori_loop(0, _ROWS, gather_chunk, jnp.int32(0))

    psub = jax.lax.broadcasted_iota(i32, (_P, _P), 0)
    plan = jax.lax.broadcasted_iota(i32, (_P, _P), 1)
    eye = (psub == plan).astype(f32)
    sub_pi = jax.lax.broadcasted_iota(i32, (_P, 1), 0)
    compact = jnp.where(sub_pi < _PRE_K, acc_s[0:_P, :], 0.0)
    compact_t = jax.lax.dot_general(
        compact, eye, (((0,), (0,)), ((), ())),
        preferred_element_type=f32, precision=jax.lax.Precision.HIGHEST)
    s_col = compact[:, 4:5]
    i_col = compact[:, 5:6]
    s_row = compact_t[4:5, :]
    i_row = compact_t[5:6, :]
    before = ((s_col > s_row) | ((s_col == s_row) & (i_col < i_row))).astype(f32)
    rank = jax.lax.dot_general(
        jnp.ones((1, _P), f32), before, (((1,), (0,)), ((), ())),
        preferred_element_type=f32, precision=jax.lax.Precision.HIGHEST)
    perm = (sub_p1 == rank).astype(f32)
    sorted_d = jax.lax.dot_general(
        perm, compact, (((1,), (0,)), ((), ())),
        preferred_element_type=f32, precision=jax.lax.Precision.HIGHEST)
    sorted_t = jax.lax.dot_general(
        sorted_d, eye, (((0,), (0,)), ((), ())),
        preferred_element_type=f32, precision=jax.lax.Precision.HIGHEST)

    x1c, y1c = sorted_d[:, 0:1], sorted_d[:, 1:2]
    x2c, y2c = sorted_d[:, 2:3], sorted_d[:, 3:4]
    x1r, y1r = sorted_t[0:1, :], sorted_t[1:2, :]
    x2r, y2r = sorted_t[2:3, :], sorted_t[3:4, :]
    area_c = jnp.maximum(x2c - x1c, 0.0) * jnp.maximum(y2c - y1c, 0.0)
    area_r = jnp.maximum(x2r - x1r, 0.0) * jnp.maximum(y2r - y1r, 0.0)
    iw = jnp.maximum(jnp.minimum(x2c, x2r) - jnp.maximum(x1c, x1r), 0.0)
    ih = jnp.maximum(jnp.minimum(y2c, y2r) - jnp.maximum(y1c, y1r), 0.0)
    inter = iw * ih
    union = area_c + area_r - inter
    iou_s[:, :] = inter / jnp.maximum(union, 1e-9)

    lane1 = jax.lax.broadcasted_iota(i32, (1, _P), 1)

    def body(bi, keep):
        base = bi * 8
        rows8 = iou_s[pl.ds(base, 8), :]
        for j in range(8):
            i = base + j
            row = rows8[j:j + 1, :]
            keep_i = jnp.sum(jnp.where(lane1 == i, keep, 0.0))
            sup = (row > _NMS_THRESH) & (lane1 > i) & (keep_i > 0.5)
            keep = jnp.where(sup, 0.0, keep)
        return keep

    keep = jax.lax.fori_loop(0, _PRE_K // 8, body, jnp.ones((1, _P), f32))

    srow_s = sorted_t[4:5, :]
    k_row = keep * (srow_s > _SCORE_THRESH).astype(f32)
    tri_p = (psub <= plan).astype(f32)
    cum_k = jax.lax.dot_general(
        k_row, tri_p, (((1,), (0,)), ((), ())), preferred_element_type=f32, precision=jax.lax.Precision.HIGHEST)
    cum_nk = jax.lax.dot_general(
        1.0 - k_row, tri_p, (((1,), (0,)), ((), ())),
        preferred_element_type=f32, precision=jax.lax.Precision.HIGHEST)
    num_k = jnp.sum(k_row)
    slot = jnp.where(k_row > 0.5, cum_k - 1.0, num_k + cum_nk - 1.0)
    out_row = jax.lax.broadcasted_iota(i32, (128, 1), 0).astype(f32)
    sel_m = (out_row == slot).astype(f32)
    k_col = jax.lax.dot_general(
        eye, k_row, (((1,), (1,)), ((), ())), preferred_element_type=f32, precision=jax.lax.Precision.HIGHEST)
    s_keep = jnp.where(k_col > 0.5, sorted_d[:, 4:5], 0.0)
    data = jnp.concatenate(
        [sorted_d[:, 0:4], s_keep, jnp.zeros((_P, 3), f32)], axis=1)
    out_ref[:, :] = jax.lax.dot_general(
        sel_m, data, (((1,), (0,)), ((), ())), preferred_element_type=f32, precision=jax.lax.Precision.HIGHEST)


def kernel(boxes, scores):
    pad = _NP - _N
    boxes_p = jnp.concatenate([boxes, jnp.zeros((pad, 4), jnp.float32)], axis=0)
    scores_p = jnp.concatenate([scores, jnp.zeros((pad,), jnp.float32)], axis=0)
    scol = scores_p[:, None]
    s2d = scores_p.reshape(_ROWS, 128)

    out = pl.pallas_call(
        _full_kernel,
        out_shape=jax.ShapeDtypeStruct((128, 8), jnp.float32),
        scratch_shapes=[
            pltpu.VMEM((_ROWS, 128), jnp.float32),
            pltpu.VMEM((_P + 128, 8), jnp.float32),
            pltpu.VMEM((_P, _P), jnp.float32),
        ],
    )(boxes_p, scol, s2d)
    return out[:_POST_K, :5]


# speedup vs baseline: 3.0044x; 2.3725x over previous
import numpy as np
import jax
import jax.numpy as jnp
from jax.experimental import pallas as pl
from jax.experimental.pallas import tpu as pltpu

_N = 20000
_NP = 20480
_ROWS = 160
_PRE_K = 1000
_POST_K = 100
_SCORE_THRESH = 0.05
_NMS_THRESH = 0.6
_P = 1024
_SENT = -1.0e30


def _full_kernel(boxes_ref, scol_ref, s2d_ref, out_ref, pos_s, acc_s):
    f32 = jnp.float32
    i32 = jnp.int32

    s2d = s2d_ref[:, :]
    masked = jnp.where(s2d > _SCORE_THRESH, s2d, _SENT)
    bits = jax.lax.bitcast_convert_type(masked, i32)
    bits = jnp.where(s2d > _SCORE_THRESH, bits, -1)

    def vsearch(_, lohi):
        lo, hi = lohi
        mid = lo + (hi - lo) // 2
        cnt = jnp.sum((bits >= mid).astype(i32))
        big = cnt >= _PRE_K
        return jnp.where(big, mid, lo), jnp.where(big, hi, mid)

    vstar, _ = jax.lax.fori_loop(
        0, 32, vsearch, (jnp.int32(-1), jnp.int32(0x40000000)))

    sub_r = jax.lax.broadcasted_iota(i32, (_ROWS, 128), 0)
    lan_r = jax.lax.broadcasted_iota(i32, (_ROWS, 128), 1)
    gidx = sub_r * 128 + lan_r
    m_need = _PRE_K - jnp.sum((bits > vstar).astype(i32))

    def isearch(_, lohi):
        lo, hi = lohi
        mid = lo + (hi - lo) // 2
        cnt = jnp.sum(((bits == vstar) & (gidx < mid)).astype(i32))
        big = cnt >= m_need
        return jnp.where(big, lo, mid), jnp.where(big, mid, hi)

    _, icut = jax.lax.fori_loop(
        0, 16, isearch, (jnp.int32(0), jnp.int32(_NP)))

    sel = (bits > vstar) | ((bits == vstar) & (gidx < icut))
    sel_f = sel.astype(f32)

    tsub = jax.lax.broadcasted_iota(i32, (128, 128), 0)
    tlan = jax.lax.broadcasted_iota(i32, (128, 128), 1)
    tri128 = (tsub <= tlan).astype(f32)
    rowcum = jax.lax.dot_general(
        sel_f, tri128, (((1,), (0,)), ((), ())),
        preferred_element_type=f32, precision=jax.lax.Precision.HIGHEST)
    rowtot = rowcum[:, 127:128]
    pos_s[:, :] = jnp.where(sel, rowcum - 1.0, -1.0)

    sub_p1 = jax.lax.broadcasted_iota(i32, (_P, 1), 0).astype(f32)
    sub_c1 = jax.lax.broadcasted_iota(i32, (128, 1), 0).astype(f32)
    sub_r1 = jax.lax.broadcasted_iota(i32, (_ROWS, 1), 0)

    def gather_chunk(c, carry):
        prow = pos_s[pl.ds(c, 1), :]
        b = boxes_ref[pl.ds(c * 128, 128), :]
        sc = scol_ref[pl.ds(c * 128, 128), :]
        msc = jnp.where(sc > _SCORE_THRESH, sc, _SENT)
        gi = sub_c1 + jnp.float32(128) * c.astype(f32)
        d = jnp.concatenate([b, msc, gi, jnp.zeros((128, 2), f32)], axis=1)
        onehot = (sub_c1 == prow).astype(f32)
        dcomp = jax.lax.dot_general(
            onehot, d, (((1,), (0,)), ((), ())),
            preferred_element_type=f32, precision=jax.lax.Precision.HIGHEST)
        base = jnp.sum(jnp.where(sub_r1 < c, rowtot, 0.0)).astype(i32)
        acc_s[pl.ds(base, 128), :] = dcomp
        return carry

    jax.lax.fori_loop(0, _ROWS, gather_chunk, jnp.int32(0))

    psub = jax.lax.broadcasted_iota(i32, (_P, _P), 0)
    plan = jax.lax.broadcasted_iota(i32, (_P, _P), 1)
    eye = (psub == plan).astype(f32)
    sub_pi = jax.lax.broadcasted_iota(i32, (_P, 1), 0)
    compact = jnp.where(sub_pi < _PRE_K, acc_s[0:_P, :], 0.0)
    compact_t = jax.lax.dot_general(
        compact, eye, (((0,), (0,)), ((), ())),
        preferred_element_type=f32, precision=jax.lax.Precision.HIGHEST)
    s_col = compact[:, 4:5]
    i_col = compact[:, 5:6]
    s_row = compact_t[4:5, :]
    i_row = compact_t[5:6, :]
    before = ((s_col > s_row) | ((s_col == s_row) & (i_col < i_row))).astype(f32)
    rank = jax.lax.dot_general(
        jnp.ones((1, _P), f32), before, (((1,), (0,)), ((), ())),
        preferred_element_type=f32, precision=jax.lax.Precision.HIGHEST)
    perm = (sub_p1 == rank).astype(f32)
    sorted_d = jax.lax.dot_general(
        perm, compact, (((1,), (0,)), ((), ())),
        preferred_element_type=f32, precision=jax.lax.Precision.HIGHEST)
    sorted_t = jax.lax.dot_general(
        sorted_d, eye, (((0,), (0,)), ((), ())),
        preferred_element_type=f32, precision=jax.lax.Precision.HIGHEST)

    x1c, y1c = sorted_d[:, 0:1], sorted_d[:, 1:2]
    x2c, y2c = sorted_d[:, 2:3], sorted_d[:, 3:4]
    x1r, y1r = sorted_t[0:1, :], sorted_t[1:2, :]
    x2r, y2r = sorted_t[2:3, :], sorted_t[3:4, :]
    area_c = jnp.maximum(x2c - x1c, 0.0) * jnp.maximum(y2c - y1c, 0.0)
    area_r = jnp.maximum(x2r - x1r, 0.0) * jnp.maximum(y2r - y1r, 0.0)
    iw = jnp.maximum(jnp.minimum(x2c, x2r) - jnp.maximum(x1c, x1r), 0.0)
    ih = jnp.maximum(jnp.minimum(y2c, y2r) - jnp.maximum(y1c, y1r), 0.0)
    inter = iw * ih
    union = area_c + area_r - inter
    iou = inter / jnp.maximum(union, 1e-9)

    supmat = ((iou > _NMS_THRESH) & (psub < plan)).astype(f32)

    def w_cond(st):
        return st[1] > 0.5

    def w_body(st):
        k, _ = st
        s = jax.lax.dot_general(
            k, supmat, (((1,), (0,)), ((), ())),
            preferred_element_type=f32)
        k_new = (s < 0.5).astype(f32)
        return k_new, jnp.sum(jnp.abs(k_new - k))

    keep, _ = jax.lax.while_loop(
        w_cond, w_body, (jnp.ones((1, _P), f32), jnp.float32(1.0)))

    srow_s = sorted_t[4:5, :]
    k_row = keep * (srow_s > _SCORE_THRESH).astype(f32)
    tri_p = (psub <= plan).astype(f32)
    cum_k = jax.lax.dot_general(
        k_row, tri_p, (((1,), (0,)), ((), ())), preferred_element_type=f32, precision=jax.lax.Precision.HIGHEST)
    cum_nk = jax.lax.dot_general(
        1.0 - k_row, tri_p, (((1,), (0,)), ((), ())),
        preferred_element_type=f32, precision=jax.lax.Precision.HIGHEST)
    num_k = jnp.sum(k_row)
    slot = jnp.where(k_row > 0.5, cum_k - 1.0, num_k + cum_nk - 1.0)
    out_row = jax.lax.broadcasted_iota(i32, (128, 1), 0).astype(f32)
    sel_m = (out_row == slot).astype(f32)
    k_col = jax.lax.dot_general(
        eye, k_row, (((1,), (1,)), ((), ())), preferred_element_type=f32, precision=jax.lax.Precision.HIGHEST)
    s_keep = jnp.where(k_col > 0.5, sorted_d[:, 4:5], 0.0)
    data = jnp.concatenate(
        [sorted_d[:, 0:4], s_keep, jnp.zeros((_P, 3), f32)], axis=1)
    out_ref[:, :] = jax.lax.dot_general(
        sel_m, data, (((1,), (0,)), ((), ())), preferred_element_type=f32, precision=jax.lax.Precision.HIGHEST)


def kernel(boxes, scores):
    pad = _NP - _N
    boxes_p = jnp.concatenate([boxes, jnp.zeros((pad, 4), jnp.float32)], axis=0)
    scores_p = jnp.concatenate([scores, jnp.zeros((pad,), jnp.float32)], axis=0)
    scol = scores_p[:, None]
    s2d = scores_p.reshape(_ROWS, 128)

    out = pl.pallas_call(
        _full_kernel,
        out_shape=jax.ShapeDtypeStruct((128, 8), jnp.float32),
        scratch_shapes=[
            pltpu.VMEM((_ROWS, 128), jnp.float32),
            pltpu.VMEM((_P + 128, 8), jnp.float32),
        ],
    )(boxes_p, scol, s2d)
    return out[:_POST_K, :5]


# speedup vs baseline: 3.0902x; 1.0286x over previous
import numpy as np
import jax
import jax.numpy as jnp
from jax.experimental import pallas as pl
from jax.experimental.pallas import tpu as pltpu

_N = 20000
_NP = 20480
_ROWS = 160
_PRE_K = 1000
_POST_K = 100
_SCORE_THRESH = 0.05
_NMS_THRESH = 0.6
_P = 1024
_SENT = -1.0e30


def _full_kernel(boxes_ref, scol_ref, s2d_ref, out_ref, pos_s, acc_s):
    f32 = jnp.float32
    i32 = jnp.int32

    s2d = s2d_ref[:, :]
    masked = jnp.where(s2d > _SCORE_THRESH, s2d, _SENT)
    bits = jax.lax.bitcast_convert_type(masked, i32)
    bits = jnp.where(s2d > _SCORE_THRESH, bits, -1)

    def vsearch(_, lohi):
        lo, hi = lohi
        mid = lo + (hi - lo) // 2
        cnt = jnp.sum((bits >= mid).astype(i32))
        big = cnt >= _PRE_K
        return jnp.where(big, mid, lo), jnp.where(big, hi, mid)

    vstar, _ = jax.lax.fori_loop(
        0, 32, vsearch, (jnp.int32(-1), jnp.int32(0x40000000)))

    sub_r = jax.lax.broadcasted_iota(i32, (_ROWS, 128), 0)
    lan_r = jax.lax.broadcasted_iota(i32, (_ROWS, 128), 1)
    gidx = sub_r * 128 + lan_r
    m_need = _PRE_K - jnp.sum((bits > vstar).astype(i32))

    def isearch(_, lohi):
        lo, hi = lohi
        mid = lo + (hi - lo) // 2
        cnt = jnp.sum(((bits == vstar) & (gidx < mid)).astype(i32))
        big = cnt >= m_need
        return jnp.where(big, lo, mid), jnp.where(big, mid, hi)

    _, icut = jax.lax.fori_loop(
        0, 16, isearch, (jnp.int32(0), jnp.int32(_NP)))

    sel = (bits > vstar) | ((bits == vstar) & (gidx < icut))
    sel_f = sel.astype(f32)

    tsub = jax.lax.broadcasted_iota(i32, (128, 128), 0)
    tlan = jax.lax.broadcasted_iota(i32, (128, 128), 1)
    tri128 = (tsub <= tlan).astype(f32)
    rowcum = jax.lax.dot_general(
        sel_f, tri128, (((1,), (0,)), ((), ())),
        preferred_element_type=f32, precision=jax.lax.Precision.DEFAULT)
    rowtot = rowcum[:, 127:128]
    pos_s[:, :] = jnp.where(sel, rowcum - 1.0, -1.0)

    sub_p1 = jax.lax.broadcasted_iota(i32, (_P, 1), 0).astype(f32)
    sub_c1 = jax.lax.broadcasted_iota(i32, (128, 1), 0).astype(f32)
    sub_r1 = jax.lax.broadcasted_iota(i32, (_ROWS, 1), 0)

    def gather_chunk(c, carry):
        prow = pos_s[pl.ds(c, 1), :]
        b = boxes_ref[pl.ds(c * 128, 128), :]
        sc = scol_ref[pl.ds(c * 128, 128), :]
        msc = jnp.where(sc > _SCORE_THRESH, sc, _SENT)
        gi = sub_c1 + jnp.float32(128) * c.astype(f32)
        d = jnp.concatenate([b, msc, gi, jnp.zeros((128, 2), f32)], axis=1)
        onehot = (sub_c1 == prow).astype(f32)
        dcomp = jax.lax.dot_general(
            onehot, d, (((1,), (0,)), ((), ())),
            preferred_element_type=f32, precision=jax.lax.Precision.HIGHEST)
        base = jnp.sum(jnp.where(sub_r1 < c, rowtot, 0.0)).astype(i32)
        acc_s[pl.ds(base, 128), :] = dcomp
        return carry

    jax.lax.fori_loop(0, _ROWS, gather_chunk, jnp.int32(0))

    psub = jax.lax.broadcasted_iota(i32, (_P, _P), 0)
    plan = jax.lax.broadcasted_iota(i32, (_P, _P), 1)
    eye = (psub == plan).astype(f32)
    sub_pi = jax.lax.broadcasted_iota(i32, (_P, 1), 0)
    compact = jnp.where(sub_pi < _PRE_K, acc_s[0:_P, :], 0.0)
    compact_t = jax.lax.dot_general(
        compact, eye, (((0,), (0,)), ((), ())),
        preferred_element_type=f32, precision=jax.lax.Precision.HIGHEST)
    s_col = compact[:, 4:5]
    i_col = compact[:, 5:6]
    s_row = compact_t[4:5, :]
    i_row = compact_t[5:6, :]
    before = ((s_col > s_row) | ((s_col == s_row) & (i_col < i_row))).astype(f32)
    rank = jax.lax.dot_general(
        jnp.ones((1, _P), f32), before, (((1,), (0,)), ((), ())),
        preferred_element_type=f32, precision=jax.lax.Precision.DEFAULT)
    perm = (sub_p1 == rank).astype(f32)
    sorted_d = jax.lax.dot_general(
        perm, compact, (((1,), (0,)), ((), ())),
        preferred_element_type=f32, precision=jax.lax.Precision.HIGHEST)
    sorted_t = jax.lax.dot_general(
        sorted_d, eye, (((0,), (0,)), ((), ())),
        preferred_element_type=f32, precision=jax.lax.Precision.HIGHEST)

    x1c, y1c = sorted_d[:, 0:1], sorted_d[:, 1:2]
    x2c, y2c = sorted_d[:, 2:3], sorted_d[:, 3:4]
    x1r, y1r = sorted_t[0:1, :], sorted_t[1:2, :]
    x2r, y2r = sorted_t[2:3, :], sorted_t[3:4, :]
    area_c = jnp.maximum(x2c - x1c, 0.0) * jnp.maximum(y2c - y1c, 0.0)
    area_r = jnp.maximum(x2r - x1r, 0.0) * jnp.maximum(y2r - y1r, 0.0)
    iw = jnp.maximum(jnp.minimum(x2c, x2r) - jnp.maximum(x1c, x1r), 0.0)
    ih = jnp.maximum(jnp.minimum(y2c, y2r) - jnp.maximum(y1c, y1r), 0.0)
    inter = iw * ih
    union = area_c + area_r - inter
    iou = inter / jnp.maximum(union, 1e-9)

    supmat = ((iou > _NMS_THRESH) & (psub < plan)).astype(f32)

    def w_cond(st):
        return st[1] > 0.5

    def w_body(st):
        k, _ = st
        s = jax.lax.dot_general(
            k, supmat, (((1,), (0,)), ((), ())),
            preferred_element_type=f32)
        k_new = (s < 0.5).astype(f32)
        return k_new, jnp.sum(jnp.abs(k_new - k))

    keep, _ = jax.lax.while_loop(
        w_cond, w_body, (jnp.ones((1, _P), f32), jnp.float32(1.0)))

    srow_s = sorted_t[4:5, :]
    k_row = keep * (srow_s > _SCORE_THRESH).astype(f32)
    tri_p = (psub <= plan).astype(f32)
    cum_k = jax.lax.dot_general(
        k_row, tri_p, (((1,), (0,)), ((), ())), preferred_element_type=f32, precision=jax.lax.Precision.DEFAULT)
    cum_nk = jax.lax.dot_general(
        1.0 - k_row, tri_p, (((1,), (0,)), ((), ())),
        preferred_element_type=f32, precision=jax.lax.Precision.DEFAULT)
    num_k = jnp.sum(k_row)
    slot = jnp.where(k_row > 0.5, cum_k - 1.0, num_k + cum_nk - 1.0)
    out_row = jax.lax.broadcasted_iota(i32, (128, 1), 0).astype(f32)
    sel_m = (out_row == slot).astype(f32)
    k_col = jax.lax.dot_general(
        eye, k_row, (((1,), (1,)), ((), ())), preferred_element_type=f32, precision=jax.lax.Precision.DEFAULT)
    s_keep = jnp.where(k_col > 0.5, sorted_d[:, 4:5], 0.0)
    data = jnp.concatenate(
        [sorted_d[:, 0:4], s_keep, jnp.zeros((_P, 3), f32)], axis=1)
    out_ref[:, :] = jax.lax.dot_general(
        sel_m, data, (((1,), (0,)), ((), ())), preferred_element_type=f32, precision=jax.lax.Precision.HIGHEST)


def kernel(boxes, scores):
    pad = _NP - _N
    boxes_p = jnp.concatenate([boxes, jnp.zeros((pad, 4), jnp.float32)], axis=0)
    scores_p = jnp.concatenate([scores, jnp.zeros((pad,), jnp.float32)], axis=0)
    scol = scores_p[:, None]
    s2d = scores_p.reshape(_ROWS, 128)

    out = pl.pallas_call(
        _full_kernel,
        out_shape=jax.ShapeDtypeStruct((128, 8), jnp.float32),
        scratch_shapes=[
            pltpu.VMEM((_ROWS, 128), jnp.float32),
            pltpu.VMEM((_P + 128, 8), jnp.float32),
        ],
    )(boxes_p, scol, s2d)
    return out[:_POST_K, :5]
